# baseline (speedup 1.0000x reference)
import functools
import math

import jax
import jax.numpy as jnp
import numpy as np
from jax import lax
from jax.experimental import pallas as pl
from jax.experimental.pallas import tpu as pltpu
from jax.experimental.pallas import tpu_sc as plsc

_D_MODEL = 768
_MAX_LEN = 2048

_NC = 2
_NS = 16
_NW = _NC * _NS


def _positional_encoding_np(max_len, d_model):
    pos = np.arange(max_len, dtype=np.float32)[:, None]
    div = np.exp(
        np.arange(0, d_model, 2, dtype=np.float32) * (-math.log(10000.0) / d_model)
    )
    pe = np.zeros((max_len, d_model), dtype=np.float32)
    pe[:, 0::2] = np.sin(pos * div)
    pe[:, 1::2] = np.cos(pos * div)
    return pe


_PE = _positional_encoding_np(_MAX_LEN, _D_MODEL)


@functools.partial(jax.jit, static_argnums=(3, 4, 5))
def _embed(x_flat, pe, table, seq_len, chunk, nchunk):
    d = table.shape[1]
    nb = _NW * nchunk * chunk
    bpw = nchunk * chunk
    scale = np.float32(math.sqrt(d))
    nvec = d // 16

    mesh = plsc.VectorSubcoreMesh(core_axis_name="c", subcore_axis_name="s")

    @functools.partial(
        pl.kernel,
        out_type=jax.ShapeDtypeStruct((nb, d), jnp.float32),
        mesh=mesh,
        scratch_types=[
            pltpu.VMEM((nchunk, chunk), jnp.int32),
            pltpu.VMEM((chunk, d), jnp.float32),
            pltpu.VMEM((chunk, d), jnp.float32),
            pltpu.SemaphoreType.DMA,
        ],
    )
    def body(x_ref, pe_ref, tab_ref, out_ref, idx_v, tok_v, pe_v, sem):
        wid = lax.axis_index("s") * _NC + lax.axis_index("c")
        base = wid * bpw
        pos0 = lax.rem(base, seq_len)
        pltpu.sync_copy(x_ref.at[wid], idx_v)
        for c in range(nchunk):
            pltpu.async_copy(tab_ref.at[idx_v.at[c]], tok_v, sem).wait()
            pltpu.sync_copy(pe_ref.at[pl.ds(pos0 + c * chunk, chunk)], pe_v)

            def row(r, carry):
                for j in range(nvec):
                    sl = pl.ds(j * 16, 16)
                    tok_v[r, sl] = tok_v[r, sl] * scale + pe_v[r, sl]
                return carry

            lax.fori_loop(0, chunk, row, 0)
            pltpu.sync_copy(tok_v, out_ref.at[pl.ds(base + c * chunk, chunk)])

    return body(x_flat, pe, table)


def kernel(x, table):
    batch, seq_len = x.shape
    d = table.shape[1]
    nb = batch * seq_len
    chunk = 64
    nchunk = nb // (_NW * chunk)
    x_flat = x.reshape(_NW, nchunk, chunk).astype(jnp.int32)
    pe = jnp.asarray(_PE[:seq_len])
    out = _embed(x_flat, pe, table, seq_len, chunk, nchunk)
    return out.reshape(batch, seq_len, d)


# speedup vs baseline: 1.0592x; 1.0592x over previous
import functools
import math

import jax
import jax.numpy as jnp
import numpy as np
from jax import lax
from jax.experimental import pallas as pl
from jax.experimental.pallas import tpu as pltpu
from jax.experimental.pallas import tpu_sc as plsc

_D_MODEL = 768
_MAX_LEN = 2048

_NC = 2
_NS = 16
_NW = _NC * _NS


def _positional_encoding_np(max_len, d_model):
    pos = np.arange(max_len, dtype=np.float32)[:, None]
    div = np.exp(
        np.arange(0, d_model, 2, dtype=np.float32) * (-math.log(10000.0) / d_model)
    )
    pe = np.zeros((max_len, d_model), dtype=np.float32)
    pe[:, 0::2] = np.sin(pos * div)
    pe[:, 1::2] = np.cos(pos * div)
    return pe


_PE = _positional_encoding_np(_MAX_LEN, _D_MODEL)

_CHUNK = 32
_POS_PER_W = 64


@functools.partial(jax.jit, static_argnums=(3, 4))
def _embed(xt, pe, table, batch, seq_len):
    d = table.shape[1]
    nb = batch * seq_len
    nchunk = batch * (_POS_PER_W // _CHUNK)
    halves = _POS_PER_W // _CHUNK
    scale = np.float32(math.sqrt(d))
    nvec = d // 16

    mesh = plsc.VectorSubcoreMesh(core_axis_name="c", subcore_axis_name="s")

    @functools.partial(
        pl.kernel,
        out_type=jax.ShapeDtypeStruct((nb, d), jnp.float32),
        mesh=mesh,
        scratch_types=[
            pltpu.VMEM((nchunk, _CHUNK), jnp.int32),
            pltpu.VMEM((_POS_PER_W, d), jnp.float32),
            pltpu.VMEM((_CHUNK, d), jnp.float32),
            pltpu.VMEM((_CHUNK, d), jnp.float32),
            pltpu.VMEM((_CHUNK, d), jnp.float32),
            pltpu.SemaphoreType.DMA,
            pltpu.SemaphoreType.DMA,
            pltpu.SemaphoreType.DMA,
            pltpu.SemaphoreType.DMA,
            pltpu.SemaphoreType.DMA,
            pltpu.SemaphoreType.DMA,
        ],
    )
    def body(x_ref, pe_ref, tab_ref, out_ref, idx_v, pe_v, t0, t1, t2,
             g0, g1, g2, o0, o1, o2):
        toks = (t0, t1, t2)
        gsem = (g0, g1, g2)
        osem = (o0, o1, o2)
        wid = lax.axis_index("s") * _NC + lax.axis_index("c")
        p0 = wid * _POS_PER_W
        pltpu.sync_copy(x_ref.at[wid], idx_v)
        pltpu.sync_copy(pe_ref.at[pl.ds(p0, _POS_PER_W)], pe_v)

        ga = [None, None, None]
        oc = [None, None, None]
        ga[0] = pltpu.async_copy(tab_ref.at[idx_v.at[0]], toks[0], gsem[0])
        ga[1] = pltpu.async_copy(tab_ref.at[idx_v.at[1]], toks[1], gsem[1])

        def make_row(buf, h):
            def row(r, carry):
                for j in range(nvec):
                    sl = pl.ds(j * 16, 16)
                    buf[r, sl] = buf[r, sl] * scale + pe_v[h * _CHUNK + r, sl]
                return carry
            return row

        for c in range(nchunk):
            a = c % 3
            b, h = divmod(c, halves)
            ga[a].wait()
            lax.fori_loop(0, _CHUNK, make_row(toks[a], h), 0)
            oc[a] = pltpu.async_copy(
                toks[a],
                out_ref.at[pl.ds(p0 + b * seq_len + h * _CHUNK, _CHUNK)],
                osem[a],
            )
            nxt = c + 2
            if nxt < nchunk:
                nb_ = nxt % 3
                if oc[nb_] is not None:
                    oc[nb_].wait()
                ga[nb_] = pltpu.async_copy(
                    tab_ref.at[idx_v.at[nxt]], toks[nb_], gsem[nb_]
                )

        for a in ((nchunk - 3) % 3, (nchunk - 2) % 3, (nchunk - 1) % 3):
            oc[a].wait()

    return body(xt, pe, table)


def kernel(x, table):
    batch, seq_len = x.shape
    d = table.shape[1]
    halves = _POS_PER_W // _CHUNK
    xt = (
        x.astype(jnp.int32)
        .reshape(batch, _NW, halves, _CHUNK)
        .transpose(1, 0, 2, 3)
        .reshape(_NW, batch * halves, _CHUNK)
    )
    pe = jnp.asarray(_PE[:seq_len])
    out = _embed(xt, pe, table, batch, seq_len)
    return out.reshape(batch, seq_len, d)


# speedup vs baseline: 1.3495x; 1.2740x over previous
import functools
import math

import jax
import jax.numpy as jnp
import numpy as np
from jax import lax
from jax.experimental import pallas as pl
from jax.experimental.pallas import tpu as pltpu
from jax.experimental.pallas import tpu_sc as plsc

_D_MODEL = 768
_MAX_LEN = 2048

_NC = 2
_NS = 16
_NW = _NC * _NS


def _positional_encoding_np(max_len, d_model):
    pos = np.arange(max_len, dtype=np.float32)[:, None]
    div = np.exp(
        np.arange(0, d_model, 2, dtype=np.float32) * (-math.log(10000.0) / d_model)
    )
    pe = np.zeros((max_len, d_model), dtype=np.float32)
    pe[:, 0::2] = np.sin(pos * div)
    pe[:, 1::2] = np.cos(pos * div)
    return pe


_PE = _positional_encoding_np(_MAX_LEN, _D_MODEL)

_CHUNK = 32
_POS_PER_W = 64
_NBUF = 3


@functools.partial(jax.jit, static_argnums=(3, 4))
def _embed(x, pe, table, batch, seq_len):
    d = table.shape[1]
    nb = batch * seq_len
    halves = _POS_PER_W // _CHUNK
    nchunk = batch * halves
    scale = np.float32(math.sqrt(d))
    nvec = d // 16

    mesh = plsc.VectorSubcoreMesh(core_axis_name="c", subcore_axis_name="s")

    @functools.partial(
        pl.kernel,
        out_type=jax.ShapeDtypeStruct((nb, d), jnp.float32),
        mesh=mesh,
        scratch_types=[
            [pltpu.VMEM((_CHUNK,), jnp.int32) for _ in range(batch * halves)],
            pltpu.VMEM((_CHUNK, d), jnp.float32),
            pltpu.VMEM((_CHUNK, d), jnp.float32),
            pltpu.VMEM((_CHUNK, d), jnp.float32),
            pltpu.VMEM((_CHUNK, d), jnp.float32),
            pltpu.VMEM((_CHUNK, d), jnp.float32),
            pltpu.SemaphoreType.DMA,
            pltpu.SemaphoreType.DMA,
            pltpu.SemaphoreType.DMA,
            pltpu.SemaphoreType.DMA,
            pltpu.SemaphoreType.DMA,
            pltpu.SemaphoreType.DMA,
            pltpu.SemaphoreType.DMA,
        ],
    )
    def body(x_ref, pe_ref, tab_ref, out_ref, idx_v, pe0, pe1, t0, t1, t2,
             gs0, gs1, gs2, os0, os1, os2, psem):
        toks = (t0, t1, t2)
        pes = (pe0, pe1)
        gsem = (gs0, gs1, gs2)
        osem = (os0, os1, os2)
        wid = lax.axis_index("s") * _NC + lax.axis_index("c")
        p0 = wid * _POS_PER_W
        for c in range(nchunk):
            h, b = divmod(c, batch)
            pltpu.sync_copy(
                x_ref.at[b, pl.ds(p0 + h * _CHUNK, _CHUNK)], idx_v[c]
            )
        pec0 = pltpu.async_copy(pe_ref.at[pl.ds(p0, _CHUNK)], pe0, psem)
        pec1 = pltpu.async_copy(pe_ref.at[pl.ds(p0 + _CHUNK, _CHUNK)], pe1, psem)

        def chunk_bh(c):
            h, b = divmod(c, batch)
            return h, b

        def gather(c, buf):
            return pltpu.async_copy(
                tab_ref.at[idx_v[c]], toks[buf], gsem[buf]
            )

        ga = [None] * _NBUF
        oc = [None] * _NBUF
        ga[0] = gather(0, 0)
        ga[1] = gather(1, 1)
        pec0.wait()
        pec1.wait()

        def make_row(buf, h):
            def row(r, carry):
                for j in range(nvec):
                    sl = pl.ds(j * 16, 16)
                    buf[r, sl] = buf[r, sl] * scale + pes[h][r, sl]
                return carry
            return row

        for c in range(nchunk):
            a = c % _NBUF
            h, b = chunk_bh(c)
            ga[a].wait()
            nxt = c + _NBUF - 1
            if nxt < nchunk:
                nb_ = nxt % _NBUF
                if oc[nb_] is not None:
                    oc[nb_].wait()
                ga[nb_] = gather(nxt, nb_)
            lax.fori_loop(0, _CHUNK, make_row(toks[a], h), 0)
            oc[a] = pltpu.async_copy(
                toks[a],
                out_ref.at[pl.ds(b * seq_len + p0 + h * _CHUNK, _CHUNK)],
                osem[a],
            )

        for a in range(_NBUF):
            oc[(nchunk - _NBUF + a) % _NBUF].wait()

    return body(x, pe, table)


def kernel(x, table):
    batch, seq_len = x.shape
    d = table.shape[1]
    pe = jnp.asarray(_PE[:seq_len])
    out = _embed(x.astype(jnp.int32), pe, table, batch, seq_len)
    return out.reshape(batch, seq_len, d)
